# baseline (speedup 1.0000x reference)
import functools

import jax
import jax.numpy as jnp
from jax import lax
from jax.experimental import pallas as pl
from jax.experimental.pallas import tpu as pltpu
from jax.experimental.pallas import tpu_sc as plsc

N_NODES = 10000
N_EDGES = 160000
IN_F = 256
OUT_F = 256
HALF = OUT_F // 2
CHUNK = 50
EDGES_PER_TILE = N_EDGES // 16
CHUNKS_PER_TILE = EDGES_PER_TILE // CHUNK
IDX_HALVES = 2
CHUNKS_PER_HALF = CHUNKS_PER_TILE // IDX_HALVES
N_SC = 2
N_SUB = 16
ROWS_PER_TILE = 624
TAIL_ROWS = N_NODES - N_SUB * ROWS_PER_TILE
ROW_BLK = 1000
N_ROW_BLKS = N_NODES // ROW_BLK


def _mm_body(h_ref, wh_ref, ng_ref, xg_ref):
    xg_ref[...] = jnp.dot(h_ref[...], wh_ref[...],
                          preferred_element_type=jnp.float32) * ng_ref[...]


def _matmul_one(h, wh, ng):
    return pl.pallas_call(
        _mm_body,
        grid=(N_ROW_BLKS, 2),
        in_specs=[
            pl.BlockSpec((ROW_BLK, IN_F), lambda i, j: (i, 0)),
            pl.BlockSpec((IN_F, HALF), lambda i, j: (0, j)),
            pl.BlockSpec((ROW_BLK, 1), lambda i, j: (i, 0)),
        ],
        out_specs=pl.BlockSpec((ROW_BLK, HALF), lambda i, j: (j * N_ROW_BLKS + i, 0)),
        out_shape=jax.ShapeDtypeStruct((N_SC * N_NODES, HALF), jnp.float32),
    )(h, wh, ng)


_SC_MESH = plsc.VectorSubcoreMesh(core_axis_name="c", subcore_axis_name="s")


@functools.partial(
    pl.kernel,
    out_type=jax.ShapeDtypeStruct((N_SC * N_NODES, HALF), jnp.float32),
    mesh=_SC_MESH,
    scratch_types=[
        pltpu.VMEM_SHARED((N_NODES, HALF), jnp.float32),
        pltpu.VMEM((CHUNKS_PER_HALF, CHUNK), jnp.int32),
        pltpu.VMEM((CHUNKS_PER_HALF, CHUNK), jnp.int32),
        pltpu.VMEM((CHUNK, HALF), jnp.float32),
        pltpu.VMEM((CHUNK, HALF), jnp.float32),
        pltpu.SemaphoreType.DMA,
        pltpu.SemaphoreType.DMA,
    ],
)
def _prop_one(x_hbm, src_hbm, dst_hbm, zeros_hbm, out_hbm,
              accum, src_v, dst_v, rows_a, rows_b, sem_a, sem_b):
    c = lax.axis_index("c")
    s = lax.axis_index("s")
    row_base = s * ROWS_PER_TILE

    pltpu.sync_copy(zeros_hbm, accum.at[pl.ds(row_base, ROWS_PER_TILE)])

    @pl.when(s == N_SUB - 1)
    def _():
        pltpu.sync_copy(zeros_hbm.at[pl.ds(0, TAIL_ROWS)],
                        accum.at[pl.ds(N_SUB * ROWS_PER_TILE, TAIL_ROWS)])

    plsc.subcore_barrier()

    for hh in range(IDX_HALVES):
        pltpu.sync_copy(src_hbm.at[c, s, hh], src_v)
        pltpu.sync_copy(dst_hbm.at[s, hh], dst_v)
        pltpu.async_copy(x_hbm.at[src_v.at[0]], rows_a, sem_a)

        def pair_body(t, carry):
            k0 = 2 * t
            pltpu.make_async_copy(x_hbm.at[src_v.at[k0]], rows_a, sem_a).wait()
            pltpu.async_copy(x_hbm.at[src_v.at[k0 + 1]], rows_b, sem_b)
            pltpu.sync_copy(rows_a, accum.at[dst_v.at[k0]], add=True)
            pltpu.make_async_copy(x_hbm.at[src_v.at[k0 + 1]], rows_b,
                                  sem_b).wait()

            @pl.when(t < CHUNKS_PER_HALF // 2 - 1)
            def _():
                pltpu.async_copy(x_hbm.at[src_v.at[k0 + 2]], rows_a, sem_a)

            pltpu.sync_copy(rows_b, accum.at[dst_v.at[k0 + 1]], add=True)
            return carry

        lax.fori_loop(0, CHUNKS_PER_HALF // 2, pair_body, 0)
    plsc.subcore_barrier()
    pltpu.sync_copy(accum.at[pl.ds(row_base, ROWS_PER_TILE)],
                    out_hbm.at[pl.ds(c * N_NODES + row_base, ROWS_PER_TILE)])

    @pl.when(s == N_SUB - 1)
    def _():
        pltpu.sync_copy(
            accum.at[pl.ds(N_SUB * ROWS_PER_TILE, TAIL_ROWS)],
            out_hbm.at[pl.ds(c * N_NODES + N_SUB * ROWS_PER_TILE, TAIL_ROWS)])


def _c1_body(g0_ref, g1_ref, ng_ref, bh_ref, p_ref):
    g = jnp.concatenate([g0_ref[...], g1_ref[...]], axis=1)
    p_ref[...] = g * ng_ref[...] + bh_ref[...]


def _c1_call(aggg, ng, bh2):
    return pl.pallas_call(
        _c1_body,
        grid=(N_ROW_BLKS,),
        in_specs=[
            pl.BlockSpec((ROW_BLK, HALF), lambda i: (i, 0)),
            pl.BlockSpec((ROW_BLK, HALF), lambda i: (N_ROW_BLKS + i, 0)),
            pl.BlockSpec((ROW_BLK, 1), lambda i: (i, 0)),
            pl.BlockSpec((1, OUT_F), lambda i: (0, 0)),
        ],
        out_specs=pl.BlockSpec((ROW_BLK, OUT_F), lambda i: (i, 0)),
        out_shape=jax.ShapeDtypeStruct((N_NODES, OUT_F), jnp.float32),
    )(aggg, aggg, ng, bh2)


def _c2_body(p_ref, f0_ref, f1_ref, nf_ref, bs_ref, out_ref):
    f = jnp.concatenate([f0_ref[...], f1_ref[...]], axis=1) * nf_ref[...]
    out_ref[...] = jnp.maximum(p_ref[...] + f + bs_ref[...], 0.0)


def _c2_call(p, aggf, nf, bs2):
    return pl.pallas_call(
        _c2_body,
        grid=(N_ROW_BLKS,),
        in_specs=[
            pl.BlockSpec((ROW_BLK, OUT_F), lambda i: (i, 0)),
            pl.BlockSpec((ROW_BLK, HALF), lambda i: (i, 0)),
            pl.BlockSpec((ROW_BLK, HALF), lambda i: (N_ROW_BLKS + i, 0)),
            pl.BlockSpec((ROW_BLK, 1), lambda i: (i, 0)),
            pl.BlockSpec((1, OUT_F), lambda i: (0, 0)),
        ],
        out_specs=pl.BlockSpec((ROW_BLK, OUT_F), lambda i: (i, 0)),
        out_shape=jax.ShapeDtypeStruct((N_NODES, OUT_F), jnp.float32),
    )(p, aggf, aggf, nf, bs2)


def kernel(h, s, edge_index_g, edge_index_f, norm_g, norm_f, wh, ws, bh, bs):
    eg = edge_index_g.astype(jnp.int32)
    ef = edge_index_f.astype(jnp.int32)
    srcg = jnp.stack([eg[0], eg[0] + N_NODES]).reshape(
        N_SC, N_SUB, IDX_HALVES, CHUNKS_PER_HALF, CHUNK)
    srcf = jnp.stack([ef[0], ef[0] + N_NODES]).reshape(
        N_SC, N_SUB, IDX_HALVES, CHUNKS_PER_HALF, CHUNK)
    dstg = eg[1].reshape(N_SUB, IDX_HALVES, CHUNKS_PER_HALF, CHUNK)
    dstf = ef[1].reshape(N_SUB, IDX_HALVES, CHUNKS_PER_HALF, CHUNK)
    zeros = jnp.zeros((ROWS_PER_TILE, HALF), jnp.float32)

    xg = _matmul_one(h, wh, norm_g)
    aggg = _prop_one(xg, srcg, dstg, zeros)
    xf = _matmul_one(s, ws, norm_f)
    aggf = _prop_one(xf, srcf, dstf, zeros)
    p = _c1_call(aggg, norm_g, bh.reshape(1, OUT_F))
    return _c2_call(p, aggf, norm_f, bs.reshape(1, OUT_F))


# speedup vs baseline: 1.4237x; 1.4237x over previous
import functools

import jax
import jax.numpy as jnp
from jax import lax
from jax.experimental import pallas as pl
from jax.experimental.pallas import tpu as pltpu
from jax.experimental.pallas import tpu_sc as plsc

N_NODES = 10000
N_EDGES = 160000
IN_F = 256
OUT_F = 256
HALF = OUT_F // 2
CHUNK = 100
EDGES_PER_TILE = N_EDGES // 16
CHUNKS_PER_TILE = EDGES_PER_TILE // CHUNK
IDX_STAGES = 4
STAGE_CHUNKS = CHUNKS_PER_TILE // IDX_STAGES
TRIPLES = STAGE_CHUNKS // 3
N_SC = 2
N_SUB = 16
ROWS_PER_TILE = 624
TAIL_ROWS = N_NODES - N_SUB * ROWS_PER_TILE
ROW_BLK = 1000
N_ROW_BLKS = N_NODES // ROW_BLK


def _mm_body(h_ref, wh_ref, ng_ref, xg_ref):
    xg_ref[...] = jnp.dot(h_ref[...], wh_ref[...],
                          preferred_element_type=jnp.float32) * ng_ref[...]


def _matmul_one(h, wh, ng):
    return pl.pallas_call(
        _mm_body,
        grid=(N_ROW_BLKS, 2),
        in_specs=[
            pl.BlockSpec((ROW_BLK, IN_F), lambda i, j: (i, 0)),
            pl.BlockSpec((IN_F, HALF), lambda i, j: (0, j)),
            pl.BlockSpec((ROW_BLK, 1), lambda i, j: (i, 0)),
        ],
        out_specs=pl.BlockSpec((ROW_BLK, HALF), lambda i, j: (j * N_ROW_BLKS + i, 0)),
        out_shape=jax.ShapeDtypeStruct((N_SC * N_NODES, HALF), jnp.float32),
    )(h, wh, ng)


_SC_MESH = plsc.VectorSubcoreMesh(core_axis_name="c", subcore_axis_name="s")


@functools.partial(
    pl.kernel,
    out_type=jax.ShapeDtypeStruct((N_SC * N_NODES, HALF), jnp.float32),
    mesh=_SC_MESH,
    scratch_types=[
        pltpu.VMEM_SHARED((N_NODES, HALF), jnp.float32),
        pltpu.VMEM((STAGE_CHUNKS, CHUNK), jnp.int32),
        pltpu.VMEM((STAGE_CHUNKS, CHUNK), jnp.int32),
        pltpu.VMEM((CHUNK, HALF), jnp.float32),
        pltpu.VMEM((CHUNK, HALF), jnp.float32),
        pltpu.VMEM((CHUNK, HALF), jnp.float32),
        pltpu.SemaphoreType.DMA,
        pltpu.SemaphoreType.DMA,
        pltpu.SemaphoreType.DMA,
        pltpu.SemaphoreType.DMA,
        pltpu.SemaphoreType.DMA,
        pltpu.SemaphoreType.DMA,
    ],
)
def _prop_one(x_hbm, src_hbm, dst_hbm, zeros_hbm, out_hbm,
              accum, src_v, dst_v, rows_a, rows_b, rows_c,
              ga, gb, gc, sa, sb, sc_):
    c = lax.axis_index("c")
    s = lax.axis_index("s")
    row_base = s * ROWS_PER_TILE

    pltpu.sync_copy(zeros_hbm, accum.at[pl.ds(row_base, ROWS_PER_TILE)])

    @pl.when(s == N_SUB - 1)
    def _():
        pltpu.sync_copy(zeros_hbm.at[pl.ds(0, TAIL_ROWS)],
                        accum.at[pl.ds(N_SUB * ROWS_PER_TILE, TAIL_ROWS)])

    plsc.subcore_barrier()

    bufs = ((rows_a, ga, sa), (rows_b, gb, sb), (rows_c, gc, sc_))

    def gather(k, buf, gsem):
        pltpu.async_copy(x_hbm.at[src_v.at[k]], buf, gsem)

    for st in range(IDX_STAGES):
        pltpu.sync_copy(src_hbm.at[c, s, st], src_v)
        pltpu.sync_copy(dst_hbm.at[s, st], dst_v)
        for j, (buf, gsem, _) in enumerate(bufs):
            gather(j, buf, gsem)

        def wait_gather(k, buf, gsem):
            pltpu.make_async_copy(x_hbm.at[src_v.at[k]], buf, gsem).wait()

        def start_scatter(k, buf, ssem):
            pltpu.async_copy(buf, accum.at[dst_v.at[k]], ssem, add=True)

        def wait_scatter_regather(k, buf, ssem, gsem):
            pltpu.make_async_copy(buf, accum.at[dst_v.at[k]], ssem).wait()

            @pl.when(k + 3 < STAGE_CHUNKS)
            def _():
                gather(k + 3, buf, gsem)

        def triple_body(t, carry):
            k0 = 3 * t
            (buf_a, ga_, sa_), (buf_b, gb_, sb_), (buf_c, gc_, sc2) = bufs
            wait_gather(k0, buf_a, ga_)
            start_scatter(k0, buf_a, sa_)
            wait_gather(k0 + 1, buf_b, gb_)
            start_scatter(k0 + 1, buf_b, sb_)
            wait_scatter_regather(k0, buf_a, sa_, ga_)
            wait_gather(k0 + 2, buf_c, gc_)
            start_scatter(k0 + 2, buf_c, sc2)
            wait_scatter_regather(k0 + 1, buf_b, sb_, gb_)
            wait_scatter_regather(k0 + 2, buf_c, sc2, gc_)
            return carry

        lax.fori_loop(0, TRIPLES, triple_body, 0)
        k_tail = STAGE_CHUNKS - 1
        pltpu.make_async_copy(x_hbm.at[src_v.at[k_tail]], rows_a, ga).wait()
        pltpu.sync_copy(rows_a, accum.at[dst_v.at[k_tail]], add=True)

    plsc.subcore_barrier()
    pltpu.sync_copy(accum.at[pl.ds(row_base, ROWS_PER_TILE)],
                    out_hbm.at[pl.ds(c * N_NODES + row_base, ROWS_PER_TILE)])

    @pl.when(s == N_SUB - 1)
    def _():
        pltpu.sync_copy(
            accum.at[pl.ds(N_SUB * ROWS_PER_TILE, TAIL_ROWS)],
            out_hbm.at[pl.ds(c * N_NODES + N_SUB * ROWS_PER_TILE, TAIL_ROWS)])


def _c1_body(g0_ref, g1_ref, ng_ref, bh_ref, p_ref):
    g = jnp.concatenate([g0_ref[...], g1_ref[...]], axis=1)
    p_ref[...] = g * ng_ref[...] + bh_ref[...]


def _c1_call(aggg, ng, bh2):
    return pl.pallas_call(
        _c1_body,
        grid=(N_ROW_BLKS,),
        in_specs=[
            pl.BlockSpec((ROW_BLK, HALF), lambda i: (i, 0)),
            pl.BlockSpec((ROW_BLK, HALF), lambda i: (N_ROW_BLKS + i, 0)),
            pl.BlockSpec((ROW_BLK, 1), lambda i: (i, 0)),
            pl.BlockSpec((1, OUT_F), lambda i: (0, 0)),
        ],
        out_specs=pl.BlockSpec((ROW_BLK, OUT_F), lambda i: (i, 0)),
        out_shape=jax.ShapeDtypeStruct((N_NODES, OUT_F), jnp.float32),
    )(aggg, aggg, ng, bh2)


def _c2_body(p_ref, f0_ref, f1_ref, nf_ref, bs_ref, out_ref):
    f = jnp.concatenate([f0_ref[...], f1_ref[...]], axis=1) * nf_ref[...]
    out_ref[...] = jnp.maximum(p_ref[...] + f + bs_ref[...], 0.0)


def _c2_call(p, aggf, nf, bs2):
    return pl.pallas_call(
        _c2_body,
        grid=(N_ROW_BLKS,),
        in_specs=[
            pl.BlockSpec((ROW_BLK, OUT_F), lambda i: (i, 0)),
            pl.BlockSpec((ROW_BLK, HALF), lambda i: (i, 0)),
            pl.BlockSpec((ROW_BLK, HALF), lambda i: (N_ROW_BLKS + i, 0)),
            pl.BlockSpec((ROW_BLK, 1), lambda i: (i, 0)),
            pl.BlockSpec((1, OUT_F), lambda i: (0, 0)),
        ],
        out_specs=pl.BlockSpec((ROW_BLK, OUT_F), lambda i: (i, 0)),
        out_shape=jax.ShapeDtypeStruct((N_NODES, OUT_F), jnp.float32),
    )(p, aggf, aggf, nf, bs2)


def kernel(h, s, edge_index_g, edge_index_f, norm_g, norm_f, wh, ws, bh, bs):
    eg = edge_index_g.astype(jnp.int32)
    ef = edge_index_f.astype(jnp.int32)
    srcg = jnp.stack([eg[0], eg[0] + N_NODES]).reshape(
        N_SC, N_SUB, IDX_STAGES, STAGE_CHUNKS, CHUNK)
    srcf = jnp.stack([ef[0], ef[0] + N_NODES]).reshape(
        N_SC, N_SUB, IDX_STAGES, STAGE_CHUNKS, CHUNK)
    dstg = eg[1].reshape(N_SUB, IDX_STAGES, STAGE_CHUNKS, CHUNK)
    dstf = ef[1].reshape(N_SUB, IDX_STAGES, STAGE_CHUNKS, CHUNK)
    zeros = jnp.zeros((ROWS_PER_TILE, HALF), jnp.float32)

    xg = _matmul_one(h, wh, norm_g)
    aggg = _prop_one(xg, srcg, dstg, zeros)
    xf = _matmul_one(s, ws, norm_f)
    aggf = _prop_one(xf, srcf, dstf, zeros)
    p = _c1_call(aggg, norm_g, bh.reshape(1, OUT_F))
    return _c2_call(p, aggf, norm_f, bs.reshape(1, OUT_F))


# speedup vs baseline: 1.4570x; 1.0234x over previous
import functools

import jax
import jax.numpy as jnp
from jax import lax
from jax.experimental import pallas as pl
from jax.experimental.pallas import tpu as pltpu
from jax.experimental.pallas import tpu_sc as plsc

N_NODES = 10000
N_EDGES = 160000
IN_F = 256
OUT_F = 256
HALF = OUT_F // 2
CHUNK = 125
EDGES_PER_TILE = N_EDGES // 16
CHUNKS_PER_TILE = EDGES_PER_TILE // CHUNK
IDX_STAGES = 2
STAGE_CHUNKS = CHUNKS_PER_TILE // IDX_STAGES
N_SC = 2
N_SUB = 16
ROWS_PER_TILE = 624
TAIL_ROWS = N_NODES - N_SUB * ROWS_PER_TILE
ROW_BLK = 1000
N_ROW_BLKS = N_NODES // ROW_BLK


def _mm_body(h_ref, wh_ref, ng_ref, x_ref):
    r = jnp.dot(h_ref[...], wh_ref[...], preferred_element_type=jnp.float32)
    x_ref[0] = r[:, :HALF] * ng_ref[...]
    x_ref[1] = r[:, HALF:] * ng_ref[...]


def _matmul_one(h, wh, ng):
    x = pl.pallas_call(
        _mm_body,
        grid=(N_ROW_BLKS,),
        in_specs=[
            pl.BlockSpec((ROW_BLK, IN_F), lambda i: (i, 0)),
            pl.BlockSpec((IN_F, OUT_F), lambda i: (0, 0)),
            pl.BlockSpec((ROW_BLK, 1), lambda i: (i, 0)),
        ],
        out_specs=pl.BlockSpec((2, ROW_BLK, HALF), lambda i: (0, i, 0)),
        out_shape=jax.ShapeDtypeStruct((N_SC, N_NODES, HALF), jnp.float32),
    )(h, wh, ng)
    return x.reshape(N_SC * N_NODES, HALF)


_SC_MESH = plsc.VectorSubcoreMesh(core_axis_name="c", subcore_axis_name="s")


@functools.partial(
    pl.kernel,
    out_type=jax.ShapeDtypeStruct((N_SC * N_NODES, HALF), jnp.float32),
    mesh=_SC_MESH,
    scratch_types=[
        pltpu.VMEM_SHARED((N_NODES, HALF), jnp.float32),
        pltpu.VMEM((STAGE_CHUNKS, CHUNK), jnp.int32),
        pltpu.VMEM((STAGE_CHUNKS, CHUNK), jnp.int32),
        pltpu.VMEM((CHUNK, HALF), jnp.float32),
        pltpu.VMEM((CHUNK, HALF), jnp.float32),
        pltpu.SemaphoreType.DMA,
        pltpu.SemaphoreType.DMA,
    ],
)
def _prop_one(x_hbm, src_hbm, dst_hbm, zeros_hbm, out_hbm,
              accum, src_v, dst_v, rows_a, rows_b, sem_a, sem_b):
    c = lax.axis_index("c")
    s = lax.axis_index("s")
    row_base = s * ROWS_PER_TILE

    pltpu.sync_copy(zeros_hbm, accum.at[pl.ds(row_base, ROWS_PER_TILE)])

    @pl.when(s == N_SUB - 1)
    def _():
        pltpu.sync_copy(zeros_hbm.at[pl.ds(0, TAIL_ROWS)],
                        accum.at[pl.ds(N_SUB * ROWS_PER_TILE, TAIL_ROWS)])

    plsc.subcore_barrier()

    for hh in range(IDX_STAGES):
        pltpu.sync_copy(src_hbm.at[c, s, hh], src_v)
        pltpu.sync_copy(dst_hbm.at[s, hh], dst_v)
        pltpu.async_copy(x_hbm.at[src_v.at[0]], rows_a, sem_a)

        def pair_body(t, carry):
            k0 = 2 * t
            pltpu.make_async_copy(x_hbm.at[src_v.at[k0]], rows_a, sem_a).wait()
            pltpu.async_copy(x_hbm.at[src_v.at[k0 + 1]], rows_b, sem_b)
            pltpu.sync_copy(rows_a, accum.at[dst_v.at[k0]], add=True)
            pltpu.make_async_copy(x_hbm.at[src_v.at[k0 + 1]], rows_b,
                                  sem_b).wait()

            @pl.when(t < STAGE_CHUNKS // 2 - 1)
            def _():
                pltpu.async_copy(x_hbm.at[src_v.at[k0 + 2]], rows_a, sem_a)

            pltpu.sync_copy(rows_b, accum.at[dst_v.at[k0 + 1]], add=True)
            return carry

        lax.fori_loop(0, STAGE_CHUNKS // 2, pair_body, 0)

    plsc.subcore_barrier()
    pltpu.sync_copy(accum.at[pl.ds(row_base, ROWS_PER_TILE)],
                    out_hbm.at[pl.ds(c * N_NODES + row_base, ROWS_PER_TILE)])

    @pl.when(s == N_SUB - 1)
    def _():
        pltpu.sync_copy(
            accum.at[pl.ds(N_SUB * ROWS_PER_TILE, TAIL_ROWS)],
            out_hbm.at[pl.ds(c * N_NODES + N_SUB * ROWS_PER_TILE, TAIL_ROWS)])


def _c1_body(g0_ref, g1_ref, ng_ref, bh_ref, p_ref):
    g = jnp.concatenate([g0_ref[...], g1_ref[...]], axis=1)
    p_ref[...] = g * ng_ref[...] + bh_ref[...]


def _c1_call(aggg, ng, bh2):
    return pl.pallas_call(
        _c1_body,
        grid=(N_ROW_BLKS,),
        in_specs=[
            pl.BlockSpec((ROW_BLK, HALF), lambda i: (i, 0)),
            pl.BlockSpec((ROW_BLK, HALF), lambda i: (N_ROW_BLKS + i, 0)),
            pl.BlockSpec((ROW_BLK, 1), lambda i: (i, 0)),
            pl.BlockSpec((1, OUT_F), lambda i: (0, 0)),
        ],
        out_specs=pl.BlockSpec((ROW_BLK, OUT_F), lambda i: (i, 0)),
        out_shape=jax.ShapeDtypeStruct((N_NODES, OUT_F), jnp.float32),
    )(aggg, aggg, ng, bh2)


def _c2_body(p_ref, f0_ref, f1_ref, nf_ref, bs_ref, out_ref):
    f = jnp.concatenate([f0_ref[...], f1_ref[...]], axis=1) * nf_ref[...]
    out_ref[...] = jnp.maximum(p_ref[...] + f + bs_ref[...], 0.0)


def _c2_call(p, aggf, nf, bs2):
    return pl.pallas_call(
        _c2_body,
        grid=(N_ROW_BLKS,),
        in_specs=[
            pl.BlockSpec((ROW_BLK, OUT_F), lambda i: (i, 0)),
            pl.BlockSpec((ROW_BLK, HALF), lambda i: (i, 0)),
            pl.BlockSpec((ROW_BLK, HALF), lambda i: (N_ROW_BLKS + i, 0)),
            pl.BlockSpec((ROW_BLK, 1), lambda i: (i, 0)),
            pl.BlockSpec((1, OUT_F), lambda i: (0, 0)),
        ],
        out_specs=pl.BlockSpec((ROW_BLK, OUT_F), lambda i: (i, 0)),
        out_shape=jax.ShapeDtypeStruct((N_NODES, OUT_F), jnp.float32),
    )(p, aggf, aggf, nf, bs2)


def kernel(h, s, edge_index_g, edge_index_f, norm_g, norm_f, wh, ws, bh, bs):
    eg = edge_index_g.astype(jnp.int32)
    ef = edge_index_f.astype(jnp.int32)
    srcg = jnp.stack([eg[0], eg[0] + N_NODES]).reshape(
        N_SC, N_SUB, IDX_STAGES, STAGE_CHUNKS, CHUNK)
    srcf = jnp.stack([ef[0], ef[0] + N_NODES]).reshape(
        N_SC, N_SUB, IDX_STAGES, STAGE_CHUNKS, CHUNK)
    dstg = eg[1].reshape(N_SUB, IDX_STAGES, STAGE_CHUNKS, CHUNK)
    dstf = ef[1].reshape(N_SUB, IDX_STAGES, STAGE_CHUNKS, CHUNK)
    zeros = jnp.zeros((ROWS_PER_TILE, HALF), jnp.float32)

    xg = _matmul_one(h, wh, norm_g)
    aggg = _prop_one(xg, srcg, dstg, zeros)
    xf = _matmul_one(s, ws, norm_f)
    aggf = _prop_one(xf, srcf, dstf, zeros)
    p = _c1_call(aggg, norm_g, bh.reshape(1, OUT_F))
    return _c2_call(p, aggf, norm_f, bs.reshape(1, OUT_F))


# speedup vs baseline: 1.4683x; 1.0078x over previous
import functools

import jax
import jax.numpy as jnp
from jax import lax
from jax.experimental import pallas as pl
from jax.experimental.pallas import tpu as pltpu
from jax.experimental.pallas import tpu_sc as plsc

N_NODES = 10000
N_EDGES = 160000
IN_F = 256
OUT_F = 256
HALF = OUT_F // 2
CHUNK = 125
EDGES_PER_TILE = N_EDGES // 16
CHUNKS_PER_TILE = EDGES_PER_TILE // CHUNK
IDX_STAGES = 2
STAGE_CHUNKS = CHUNKS_PER_TILE // IDX_STAGES
N_SC = 2
N_SUB = 16
ROWS_PER_TILE = 624
TAIL_ROWS = N_NODES - N_SUB * ROWS_PER_TILE
ROW_BLK = 1000
N_ROW_BLKS = N_NODES // ROW_BLK


def _mm_body(h_ref, wh_ref, ng_ref, x_ref):
    r = jnp.dot(h_ref[...], wh_ref[...], preferred_element_type=jnp.float32)
    x_ref[0] = r[:, :HALF] * ng_ref[...]
    x_ref[1] = r[:, HALF:] * ng_ref[...]


def _matmul_one(h, wh, ng):
    x = pl.pallas_call(
        _mm_body,
        grid=(N_ROW_BLKS,),
        in_specs=[
            pl.BlockSpec((ROW_BLK, IN_F), lambda i: (i, 0)),
            pl.BlockSpec((IN_F, OUT_F), lambda i: (0, 0)),
            pl.BlockSpec((ROW_BLK, 1), lambda i: (i, 0)),
        ],
        out_specs=pl.BlockSpec((2, ROW_BLK, HALF), lambda i: (0, i, 0)),
        out_shape=jax.ShapeDtypeStruct((N_SC, N_NODES, HALF), jnp.float32),
    )(h, wh, ng)
    return x.reshape(N_SC * N_NODES, HALF)


_SC_MESH = plsc.VectorSubcoreMesh(core_axis_name="c", subcore_axis_name="s")


@functools.partial(
    pl.kernel,
    out_type=jax.ShapeDtypeStruct((N_SC * N_NODES, HALF), jnp.float32),
    mesh=_SC_MESH,
    scratch_types=[
        pltpu.VMEM_SHARED((N_NODES, HALF), jnp.float32),
        pltpu.VMEM((STAGE_CHUNKS, CHUNK), jnp.int32),
        pltpu.VMEM((STAGE_CHUNKS, CHUNK), jnp.int32),
        pltpu.VMEM((CHUNK, HALF), jnp.float32),
        pltpu.VMEM((CHUNK, HALF), jnp.float32),
        pltpu.SemaphoreType.DMA,
        pltpu.SemaphoreType.DMA,
    ],
)
def _prop_one(x_hbm, src_hbm, dst_hbm, zeros_hbm, out_hbm,
              accum, src_v, dst_v, rows_a, rows_b, sem_a, sem_b):
    c = lax.axis_index("c")
    s = lax.axis_index("s")
    row_base = s * ROWS_PER_TILE

    pltpu.sync_copy(zeros_hbm, accum.at[pl.ds(row_base, ROWS_PER_TILE)])

    @pl.when(s == N_SUB - 1)
    def _():
        pltpu.sync_copy(zeros_hbm.at[pl.ds(0, TAIL_ROWS)],
                        accum.at[pl.ds(N_SUB * ROWS_PER_TILE, TAIL_ROWS)])

    plsc.subcore_barrier()

    for hh in range(IDX_STAGES):
        pltpu.sync_copy(src_hbm.at[c, s, hh], src_v)
        pltpu.sync_copy(dst_hbm.at[s, hh], dst_v)
        pltpu.async_copy(x_hbm.at[src_v.at[0]], rows_a, sem_a)

        def pair_body(t, carry):
            k0 = 2 * t
            pltpu.make_async_copy(x_hbm.at[src_v.at[k0]], rows_a, sem_a).wait()
            pltpu.async_copy(x_hbm.at[src_v.at[k0 + 1]], rows_b, sem_b)
            pltpu.sync_copy(rows_a, accum.at[dst_v.at[k0]], add=True)
            pltpu.make_async_copy(x_hbm.at[src_v.at[k0 + 1]], rows_b,
                                  sem_b).wait()

            @pl.when(t < STAGE_CHUNKS // 2 - 1)
            def _():
                pltpu.async_copy(x_hbm.at[src_v.at[k0 + 2]], rows_a, sem_a)

            pltpu.sync_copy(rows_b, accum.at[dst_v.at[k0 + 1]], add=True)
            return carry

        lax.fori_loop(0, STAGE_CHUNKS // 2, pair_body, 0)

    plsc.subcore_barrier()
    pltpu.sync_copy(accum.at[pl.ds(row_base, ROWS_PER_TILE)],
                    out_hbm.at[pl.ds(c * N_NODES + row_base, ROWS_PER_TILE)])

    @pl.when(s == N_SUB - 1)
    def _():
        pltpu.sync_copy(
            accum.at[pl.ds(N_SUB * ROWS_PER_TILE, TAIL_ROWS)],
            out_hbm.at[pl.ds(c * N_NODES + N_SUB * ROWS_PER_TILE, TAIL_ROWS)])


def _cm_body(g0_ref, g1_ref, f0_ref, f1_ref, ng_ref, nf_ref,
             bh_ref, bs_ref, out_ref):
    g = jnp.concatenate([g0_ref[...], g1_ref[...]], axis=1) * ng_ref[...]
    f = jnp.concatenate([f0_ref[...], f1_ref[...]], axis=1) * nf_ref[...]
    out_ref[...] = jnp.maximum(g + f + bh_ref[...] + bs_ref[...], 0.0)


def _cm_call(aggg, aggf, ng, nf, bh2, bs2):
    return pl.pallas_call(
        _cm_body,
        grid=(N_ROW_BLKS,),
        in_specs=[
            pl.BlockSpec((ROW_BLK, HALF), lambda i: (i, 0)),
            pl.BlockSpec((ROW_BLK, HALF), lambda i: (N_ROW_BLKS + i, 0)),
            pl.BlockSpec((ROW_BLK, HALF), lambda i: (i, 0)),
            pl.BlockSpec((ROW_BLK, HALF), lambda i: (N_ROW_BLKS + i, 0)),
            pl.BlockSpec((ROW_BLK, 1), lambda i: (i, 0)),
            pl.BlockSpec((ROW_BLK, 1), lambda i: (i, 0)),
            pl.BlockSpec((1, OUT_F), lambda i: (0, 0)),
            pl.BlockSpec((1, OUT_F), lambda i: (0, 0)),
        ],
        out_specs=pl.BlockSpec((ROW_BLK, OUT_F), lambda i: (i, 0)),
        out_shape=jax.ShapeDtypeStruct((N_NODES, OUT_F), jnp.float32),
    )(aggg, aggg, aggf, aggf, ng, nf, bh2, bs2)


def kernel(h, s, edge_index_g, edge_index_f, norm_g, norm_f, wh, ws, bh, bs):
    eg = edge_index_g.astype(jnp.int32)
    ef = edge_index_f.astype(jnp.int32)
    srcg = jnp.stack([eg[0], eg[0] + N_NODES]).reshape(
        N_SC, N_SUB, IDX_STAGES, STAGE_CHUNKS, CHUNK)
    srcf = jnp.stack([ef[0], ef[0] + N_NODES]).reshape(
        N_SC, N_SUB, IDX_STAGES, STAGE_CHUNKS, CHUNK)
    dstg = eg[1].reshape(N_SUB, IDX_STAGES, STAGE_CHUNKS, CHUNK)
    dstf = ef[1].reshape(N_SUB, IDX_STAGES, STAGE_CHUNKS, CHUNK)
    zeros = jnp.zeros((ROWS_PER_TILE, HALF), jnp.float32)

    xg = _matmul_one(h, wh, norm_g)
    aggg = _prop_one(xg, srcg, dstg, zeros)
    xf = _matmul_one(s, ws, norm_f)
    aggf = _prop_one(xf, srcf, dstf, zeros)
    return _cm_call(aggg, aggf, norm_g, norm_f,
                    bh.reshape(1, OUT_F), bs.reshape(1, OUT_F))
